# Initial kernel scaffold; baseline (speedup 1.0000x reference)
import jax
import jax.numpy as jnp
from jax.experimental import pallas as pl


def kernel(difference, weight, epoch, iteration):
    raise NotImplementedError("write your pallas kernel here")


# speedup vs baseline: 147.4372x; 147.4372x over previous
import jax
import jax.numpy as jnp
from jax.experimental import pallas as pl
from jax.experimental.pallas import tpu as pltpu

_B = 64
_N = 32768
_K = 1000
_R = 8


def _max1_kernel(cond_ref, d_ref, w_ref, o_ref):
    @pl.when(cond_ref[0] == 0)
    def _copy():
        o_ref[...] = w_ref[...]

    @pl.when(cond_ref[0] != 0)
    def _topk_mask():
        a = jnp.abs(d_ref[...])
        bits = jax.lax.bitcast_convert_type(a, jnp.int32)
        one = jnp.int32(1)

        def kth_body(i, cur):
            cand = jnp.bitwise_or(cur, jnp.left_shift(one, 30 - i))
            cnt = jnp.sum((bits >= cand).astype(jnp.int32), axis=1,
                          keepdims=True)
            return jnp.where(cnt >= _K, cand, cur)

        kth = jax.lax.fori_loop(0, 31, kth_body,
                                jnp.zeros((_R, 1), jnp.int32))

        gt = bits > kth
        need = _K - jnp.sum(gt.astype(jnp.int32), axis=1, keepdims=True)
        eq = bits == kth
        idx = jax.lax.broadcasted_iota(jnp.int32, bits.shape, 1)

        def cut_body(i, cur):
            cand = jnp.bitwise_or(cur, jnp.left_shift(one, 15 - i))
            cnt = jnp.sum((eq & (idx < cand)).astype(jnp.int32), axis=1,
                          keepdims=True)
            return jnp.where(cnt <= need, cand, cur)

        cut = jax.lax.fori_loop(0, 16, cut_body,
                                jnp.zeros((_R, 1), jnp.int32))

        sel = gt | (eq & (idx < cut))
        o_ref[...] = w_ref[...] + sel.astype(jnp.float32)


def kernel(difference, weight, epoch, iteration):
    del iteration
    epoch = jnp.asarray(epoch, jnp.int32)
    cond = ((epoch > 1000) & (epoch < 18000)
            & (epoch % 200 == 0)).astype(jnp.int32).reshape(1)

    grid = _B // _R
    out = pl.pallas_call(
        _max1_kernel,
        grid_spec=pltpu.PrefetchScalarGridSpec(
            num_scalar_prefetch=1,
            grid=(grid,),
            in_specs=[
                pl.BlockSpec(
                    (_R, _N),
                    lambda i, cond_ref: (
                        jnp.where(cond_ref[0] != 0, i, 0), 0)),
                pl.BlockSpec((_R, _N), lambda i, cond_ref: (i, 0)),
            ],
            out_specs=pl.BlockSpec((_R, _N), lambda i, cond_ref: (i, 0)),
        ),
        out_shape=jax.ShapeDtypeStruct((_B, _N), jnp.float32),
    )(cond, difference, weight)
    return out


# speedup vs baseline: 169.5928x; 1.1503x over previous
import jax
import jax.numpy as jnp
from jax.experimental import pallas as pl
from jax.experimental.pallas import tpu as pltpu

_B = 64
_N = 32768
_K = 1000
_R = 16


def _max1_kernel(cond_ref, d_ref, w_ref, o_ref):
    @pl.when(cond_ref[0] == 0)
    def _copy():
        o_ref[...] = w_ref[...]

    @pl.when(cond_ref[0] != 0)
    def _topk_mask():
        a = jnp.abs(d_ref[...])
        bits = jax.lax.bitcast_convert_type(a, jnp.int32)
        one = jnp.int32(1)

        def kth_body(i, cur):
            cand = jnp.bitwise_or(cur, jnp.left_shift(one, 30 - i))
            cnt = jnp.sum((bits >= cand).astype(jnp.int32), axis=1,
                          keepdims=True)
            return jnp.where(cnt >= _K, cand, cur)

        kth = jax.lax.fori_loop(0, 31, kth_body,
                                jnp.zeros((_R, 1), jnp.int32))

        gt = bits > kth
        need = _K - jnp.sum(gt.astype(jnp.int32), axis=1, keepdims=True)
        eq = bits == kth
        idx = jax.lax.broadcasted_iota(jnp.int32, bits.shape, 1)

        def cut_body(i, cur):
            cand = jnp.bitwise_or(cur, jnp.left_shift(one, 15 - i))
            cnt = jnp.sum((eq & (idx < cand)).astype(jnp.int32), axis=1,
                          keepdims=True)
            return jnp.where(cnt <= need, cand, cur)

        cut = jax.lax.fori_loop(0, 16, cut_body,
                                jnp.zeros((_R, 1), jnp.int32))

        sel = gt | (eq & (idx < cut))
        o_ref[...] = w_ref[...] + sel.astype(jnp.float32)


def kernel(difference, weight, epoch, iteration):
    del iteration
    epoch = jnp.asarray(epoch, jnp.int32)
    cond = ((epoch > 1000) & (epoch < 18000)
            & (epoch % 200 == 0)).astype(jnp.int32).reshape(1)

    grid = _B // _R
    out = pl.pallas_call(
        _max1_kernel,
        grid_spec=pltpu.PrefetchScalarGridSpec(
            num_scalar_prefetch=1,
            grid=(grid,),
            in_specs=[
                pl.BlockSpec(
                    (_R, _N),
                    lambda i, cond_ref: (
                        jnp.where(cond_ref[0] != 0, i, 0), 0)),
                pl.BlockSpec((_R, _N), lambda i, cond_ref: (i, 0)),
            ],
            out_specs=pl.BlockSpec((_R, _N), lambda i, cond_ref: (i, 0)),
        ),
        out_shape=jax.ShapeDtypeStruct((_B, _N), jnp.float32),
    )(cond, difference, weight)
    return out
